# baseline (speedup 1.0000x reference)
import functools

import jax
import jax.numpy as jnp
from jax import lax
from jax.experimental import pallas as pl
from jax.experimental.pallas import tpu as pltpu
from jax.experimental.pallas import tpu_sc as plsc

_N_ACT = 4
_HID = 256
_BATCH = 16384
_NC, _NS = 2, 16
_NW = _NC * _NS
_BPW = _BATCH // _NW
_CH = 256
_NCH = _BPW // _CH


def _mlp_table_body(emb_ref, w1_ref, b1_ref, w2_ref, b2_ref, out_ref):
    x = emb_ref[...]
    h = lax.dot_general(x, w1_ref[...], (((1,), (1,)), ((), ())),
                        preferred_element_type=jnp.float32) + b1_ref[...]
    h = h * jax.nn.sigmoid(h)
    out_ref[...] = lax.dot_general(h, w2_ref[...], (((1,), (1,)), ((), ())),
                                   preferred_element_type=jnp.float32) + b2_ref[...]


def _gather_body(table_hbm, idx_hbm, out_hbm, idx_v, buf, gsem):
    wid = lax.axis_index("s") * _NC + lax.axis_index("c")
    base = wid * _BPW
    pltpu.sync_copy(idx_hbm.at[pl.ds(base, _BPW)], idx_v)
    for c in range(_NCH):
        pltpu.async_copy(
            table_hbm.at[idx_v.at[pl.ds(c * _CH, _CH)]], buf, gsem
        ).wait()
        pltpu.sync_copy(buf, out_hbm.at[pl.ds(base + c * _CH, _CH)])


@functools.cache
def _make_gather():
    return pl.kernel(
        _gather_body,
        out_type=jax.ShapeDtypeStruct((_BATCH, _HID), jnp.float32),
        mesh=plsc.VectorSubcoreMesh(core_axis_name="c", subcore_axis_name="s"),
        scratch_types=[
            pltpu.VMEM((_BPW,), jnp.int32),
            pltpu.VMEM((_CH, _HID), jnp.float32),
            pltpu.SemaphoreType.DMA,
        ],
    )


def kernel(actions, emb_table, W1, b1, W2, b2):
    table = pl.pallas_call(
        _mlp_table_body,
        out_shape=jax.ShapeDtypeStruct((_N_ACT, _HID), jnp.float32),
    )(emb_table, W1, b1.reshape(1, _HID), W2, b2.reshape(1, _HID))
    return _make_gather()(table, actions)


# speedup vs baseline: 5.3470x; 5.3470x over previous
import functools

import jax
import jax.numpy as jnp
from jax import lax
from jax.experimental import pallas as pl
from jax.experimental.pallas import tpu as pltpu
from jax.experimental.pallas import tpu_sc as plsc

_N_ACT = 4
_HID = 256
_BATCH = 16384
_NC, _NS = 2, 16
_NW = _NC * _NS
_BPW = _BATCH // _NW
_CH = 128
_NCH = _BPW // _CH


def _mlp_table_body(emb_ref, w1_ref, b1_ref, w2_ref, b2_ref, out_ref):
    x = emb_ref[...]
    h = lax.dot_general(x, w1_ref[...], (((1,), (1,)), ((), ())),
                        preferred_element_type=jnp.float32) + b1_ref[...]
    h = h * jax.nn.sigmoid(h)
    out_ref[...] = lax.dot_general(h, w2_ref[...], (((1,), (1,)), ((), ())),
                                   preferred_element_type=jnp.float32) + b2_ref[...]


def _gather_body(table_hbm, idx_hbm, out_hbm, idx_v, table_v, buf0, buf1,
                 sem0, sem1):
    wid = lax.axis_index("s") * _NC + lax.axis_index("c")
    base = wid * _BPW
    pltpu.sync_copy(idx_hbm.at[pl.ds(base, _BPW)], idx_v)
    pltpu.sync_copy(table_hbm, table_v)
    bufs = (buf0, buf1)
    sems = (sem0, sem1)
    pending = [None, None]

    def make_fill(ch, buf):
        def fill(g, carry):
            av = idx_v[pl.ds(ch * _CH + g * 16, 16)]
            for j in range(16):
                a = av[j]
                r = g * 16 + j
                vals = [table_v[a, pl.ds(c * 16, 16)]
                        for c in range(_HID // 16)]
                for c in range(_HID // 16):
                    buf[r, pl.ds(c * 16, 16)] = vals[c]
            return carry
        return fill

    for ch in range(_NCH):
        slot = ch % 2
        if pending[slot] is not None:
            pending[slot].wait()
        lax.fori_loop(0, _CH // 16, make_fill(ch, bufs[slot]), 0, unroll=False)
        pending[slot] = pltpu.async_copy(
            bufs[slot], out_hbm.at[pl.ds(base + ch * _CH, _CH)], sems[slot])
    for p in pending:
        if p is not None:
            p.wait()


@functools.cache
def _make_gather():
    return pl.kernel(
        _gather_body,
        out_type=jax.ShapeDtypeStruct((_BATCH, _HID), jnp.float32),
        mesh=plsc.VectorSubcoreMesh(core_axis_name="c", subcore_axis_name="s"),
        scratch_types=[
            pltpu.VMEM((_BPW,), jnp.int32),
            pltpu.VMEM((_N_ACT, _HID), jnp.float32),
            pltpu.VMEM((_CH, _HID), jnp.float32),
            pltpu.VMEM((_CH, _HID), jnp.float32),
            pltpu.SemaphoreType.DMA,
            pltpu.SemaphoreType.DMA,
        ],
    )


def kernel(actions, emb_table, W1, b1, W2, b2):
    table = pl.pallas_call(
        _mlp_table_body,
        out_shape=jax.ShapeDtypeStruct((_N_ACT, _HID), jnp.float32),
    )(emb_table, W1, b1.reshape(1, _HID), W2, b2.reshape(1, _HID))
    return _make_gather()(table, actions)
